# Initial kernel scaffold; baseline (speedup 1.0000x reference)
import jax
import jax.numpy as jnp
from jax.experimental import pallas as pl


def kernel(x, noise):
    raise NotImplementedError("write your pallas kernel here")


# speedup vs baseline: 1.8134x; 1.8134x over previous
import functools

import jax
import jax.numpy as jnp
from jax import lax
from jax.experimental import pallas as pl
from jax.experimental.pallas import tpu as pltpu
from jax.experimental.pallas import tpu_sc as plsc

_MASKING_RATIO = 0.75


def _sort_mask_body(noise_ref, gids_ref, mask_ref):
    n, l = noise_ref.shape
    keep = gids_ref.shape[1]
    orig = noise_ref[...]
    pos = lax.broadcasted_iota(jnp.int32, (n, l), 1)
    val = orig
    key_i = pos
    k = 2
    while k <= l:
        j = k // 2
        while j >= 1:
            low = (pos & j) == 0
            pval = jnp.where(low, jnp.roll(val, -j, axis=1),
                             jnp.roll(val, j, axis=1))
            pidx = jnp.where(low, jnp.roll(key_i, -j, axis=1),
                             jnp.roll(key_i, j, axis=1))
            lt = (val < pval) | ((val == pval) & (key_i < pidx))
            up = (pos & k) == 0
            keep_small = low == up
            take_self = lt == keep_small
            val = jnp.where(take_self, val, pval)
            key_i = jnp.where(take_self, key_i, pidx)
            j //= 2
        k *= 2
    row = lax.broadcasted_iota(jnp.int32, (n, keep), 0)
    gids_ref[...] = key_i[:, :keep] + row * l
    tv = val[:, keep:keep + 1]
    ti = key_i[:, keep:keep + 1]
    mask = (orig > tv) | ((orig == tv) & (pos >= ti))
    mask_ref[...] = mask.astype(mask_ref.dtype)


def _sort_mask(noise, keep):
    n, l = noise.shape
    return pl.pallas_call(
        _sort_mask_body,
        out_shape=(
            jax.ShapeDtypeStruct((n, keep), jnp.int32),
            jax.ShapeDtypeStruct((n, l), jnp.float32),
        ),
    )(noise)


def _make_sc_gather(rows, d, chunk):
    info = plsc.get_sparse_core_info()
    nc, ns = info.num_cores, info.num_subcores
    nw = nc * ns
    per_w = rows // nw
    assert per_w % chunk == 0 and per_w % 8 == 0
    n_chunks = per_w // chunk
    mesh = plsc.VectorSubcoreMesh(core_axis_name="c", subcore_axis_name="s")

    @functools.partial(
        pl.kernel,
        mesh=mesh,
        out_type=jax.ShapeDtypeStruct((rows, d), jnp.float32),
        scratch_types=[
            pltpu.VMEM((chunk,), jnp.int32),
            pltpu.VMEM((chunk, d), jnp.float32),
            pltpu.SemaphoreType.DMA,
        ],
    )
    def gather(table_hbm, gids_hbm, out_hbm, idx_v, rows_v, sem):
        wid = lax.axis_index("s") * nc + lax.axis_index("c")
        base = wid * per_w

        def body(c, carry):
            off = base + c * chunk
            pltpu.sync_copy(gids_hbm.at[pl.ds(off, chunk)], idx_v)
            pltpu.async_copy(table_hbm.at[idx_v], rows_v, sem).wait()
            pltpu.sync_copy(rows_v, out_hbm.at[pl.ds(off, chunk)])
            return carry

        lax.fori_loop(0, n_chunks, body, 0)

    return gather


def kernel(x, noise):
    n, l, d = x.shape
    keep = int(l * (1 - _MASKING_RATIO))
    gids, mask = _sort_mask(noise, keep)
    table = x.reshape(n * l, d)
    gather = _make_sc_gather(n * keep, d, chunk=128)
    x_masked = gather(table, gids.reshape(-1))
    return x_masked.reshape(n, keep, d), mask


# speedup vs baseline: 1.8225x; 1.0051x over previous
import functools

import jax
import jax.numpy as jnp
from jax import lax
from jax.experimental import pallas as pl
from jax.experimental.pallas import tpu as pltpu
from jax.experimental.pallas import tpu_sc as plsc

_MASKING_RATIO = 0.75


def _sort_mask_body(noise_ref, gids_ref, mask_ref):
    n, l = noise_ref.shape
    keep = gids_ref.shape[1]
    orig = noise_ref[...]
    pos = lax.broadcasted_iota(jnp.int32, (n, l), 1)
    val = orig
    key_i = pos
    k = 2
    while k <= l:
        j = k // 2
        while j >= 1:
            low = (pos & j) == 0
            pval = jnp.where(low, jnp.roll(val, -j, axis=1),
                             jnp.roll(val, j, axis=1))
            pidx = jnp.where(low, jnp.roll(key_i, -j, axis=1),
                             jnp.roll(key_i, j, axis=1))
            lt = (val < pval) | ((val == pval) & (key_i < pidx))
            up = (pos & k) == 0
            keep_small = low == up
            take_self = lt == keep_small
            val = jnp.where(take_self, val, pval)
            key_i = jnp.where(take_self, key_i, pidx)
            j //= 2
        k *= 2
    row = lax.broadcasted_iota(jnp.int32, (n, keep), 0)
    gids_ref[...] = key_i[:, :keep] + row * l
    tv = val[:, keep:keep + 1]
    ti = key_i[:, keep:keep + 1]
    mask = (orig > tv) | ((orig == tv) & (pos >= ti))
    mask_ref[...] = mask.astype(mask_ref.dtype)


def _sort_mask(noise, keep):
    n, l = noise.shape
    return pl.pallas_call(
        _sort_mask_body,
        out_shape=(
            jax.ShapeDtypeStruct((n, keep), jnp.int32),
            jax.ShapeDtypeStruct((n, l), jnp.float32),
        ),
    )(noise)


def _make_sc_gather(rows, d, chunk):
    info = plsc.get_sparse_core_info()
    nc, ns = info.num_cores, info.num_subcores
    nw = nc * ns
    per_w = rows // nw
    assert per_w % chunk == 0 and per_w % 8 == 0 and chunk % 8 == 0
    n_chunks = per_w // chunk
    mesh = plsc.VectorSubcoreMesh(core_axis_name="c", subcore_axis_name="s")

    @functools.partial(
        pl.kernel,
        mesh=mesh,
        out_type=jax.ShapeDtypeStruct((rows, d), jnp.float32),
        scratch_types=[
            pltpu.VMEM((per_w,), jnp.int32),
            pltpu.VMEM((chunk, d), jnp.float32),
            pltpu.VMEM((chunk, d), jnp.float32),
            pltpu.SemaphoreType.DMA,
            pltpu.SemaphoreType.DMA,
        ],
    )
    def gather(table_hbm, gids_hbm, out_hbm, idx_all, b0, b1, s0, s1):
        wid = lax.axis_index("s") * nc + lax.axis_index("c")
        base = wid * per_w
        pltpu.sync_copy(gids_hbm.at[pl.ds(base, per_w)], idx_all)
        bufs = (b0, b1)
        sems = (s0, s1)

        def start(c):
            return pltpu.async_copy(
                table_hbm.at[idx_all.at[pl.ds(c * chunk, chunk)]],
                bufs[c % 2], sems[c % 2])

        cp = start(0)
        for c in range(n_chunks):
            nxt = start(c + 1) if c + 1 < n_chunks else None
            cp.wait()
            pltpu.sync_copy(bufs[c % 2], out_hbm.at[pl.ds(base + c * chunk, chunk)])
            cp = nxt

    return gather


def kernel(x, noise):
    n, l, d = x.shape
    keep = int(l * (1 - _MASKING_RATIO))
    gids, mask = _sort_mask(noise, keep)
    table = x.reshape(n * l, d)
    gather = _make_sc_gather(n * keep, d, chunk=64)
    x_masked = gather(table, gids.reshape(-1))
    return x_masked.reshape(n, keep, d), mask


# speedup vs baseline: 1.8378x; 1.0084x over previous
import functools

import jax
import jax.numpy as jnp
from jax import lax
from jax.experimental import pallas as pl
from jax.experimental.pallas import tpu as pltpu
from jax.experimental.pallas import tpu_sc as plsc

_MASKING_RATIO = 0.75


def _sort_mask_body(noise_ref, gids_ref, mask_ref):
    n, l = noise_ref.shape
    keep = gids_ref.shape[1]
    orig = noise_ref[...]
    pos = lax.broadcasted_iota(jnp.int32, (n, l), 1)
    val = orig
    key_i = pos
    k = 2
    while k <= l:
        j = k // 2
        while j >= 1:
            low = (pos & j) == 0
            pval = jnp.where(low, jnp.roll(val, -j, axis=1),
                             jnp.roll(val, j, axis=1))
            pidx = jnp.where(low, jnp.roll(key_i, -j, axis=1),
                             jnp.roll(key_i, j, axis=1))
            lt = (val < pval) | ((val == pval) & (key_i < pidx))
            up = (pos & k) == 0
            keep_small = low == up
            take_self = lt == keep_small
            val = jnp.where(take_self, val, pval)
            key_i = jnp.where(take_self, key_i, pidx)
            j //= 2
        k *= 2
    row = lax.broadcasted_iota(jnp.int32, (n, keep), 0)
    gids_ref[...] = key_i[:, :keep] + row * l
    tv = val[:, keep:keep + 1]
    ti = key_i[:, keep:keep + 1]
    mask = (orig > tv) | ((orig == tv) & (pos >= ti))
    mask_ref[...] = mask.astype(mask_ref.dtype)


def _sort_mask(noise, keep):
    n, l = noise.shape
    return pl.pallas_call(
        _sort_mask_body,
        out_shape=(
            jax.ShapeDtypeStruct((n, keep), jnp.int32),
            jax.ShapeDtypeStruct((n, l), jnp.float32),
        ),
    )(noise)


def _make_sc_gather(rows, d, chunk):
    info = plsc.get_sparse_core_info()
    nc, ns = info.num_cores, info.num_subcores
    nw = nc * ns
    per_w = rows // nw
    assert per_w % chunk == 0 and per_w % 8 == 0 and chunk % 8 == 0
    n_chunks = per_w // chunk
    mesh = plsc.VectorSubcoreMesh(core_axis_name="c", subcore_axis_name="s")

    @functools.partial(
        pl.kernel,
        mesh=mesh,
        out_type=jax.ShapeDtypeStruct((rows, d), jnp.float32),
        scratch_types=(
            [pltpu.VMEM((per_w,), jnp.int32)]
            + [pltpu.VMEM((chunk, d), jnp.float32) for _ in range(4)]
            + [pltpu.SemaphoreType.DMA for _ in range(8)]
        ),
    )
    def gather(table_hbm, gids_hbm, out_hbm, idx_all, *bufsem):
        nbuf = 4
        bufs = bufsem[:nbuf]
        gsem = bufsem[nbuf:2 * nbuf]
        wsem = bufsem[2 * nbuf:3 * nbuf]
        wid = lax.axis_index("s") * nc + lax.axis_index("c")
        base = wid * per_w
        pltpu.sync_copy(gids_hbm.at[pl.ds(base, per_w)], idx_all)

        def start(c):
            return pltpu.async_copy(
                table_hbm.at[idx_all.at[pl.ds(c * chunk, chunk)]],
                bufs[c % nbuf], gsem[c % nbuf])

        def wback(c):
            return pltpu.async_copy(
                bufs[c % nbuf], out_hbm.at[pl.ds(base + c * chunk, chunk)],
                wsem[c % nbuf])

        gcp = [None] * n_chunks
        wcp = [None] * n_chunks
        for c in range(min(nbuf, n_chunks)):
            gcp[c] = start(c)
        for c in range(n_chunks):
            gcp[c].wait()
            wcp[c] = wback(c)
            if c + nbuf < n_chunks:
                wcp[c].wait()
                gcp[c + nbuf] = start(c + nbuf)
        for c in range(max(0, n_chunks - nbuf), n_chunks):
            wcp[c].wait()

    return gather


def kernel(x, noise):
    n, l, d = x.shape
    keep = int(l * (1 - _MASKING_RATIO))
    gids, mask = _sort_mask(noise, keep)
    table = x.reshape(n * l, d)
    gather = _make_sc_gather(n * keep, d, chunk=32)
    x_masked = gather(table, gids.reshape(-1))
    return x_masked.reshape(n, keep, d), mask


# speedup vs baseline: 1.9074x; 1.0378x over previous
import functools

import jax
import jax.numpy as jnp
from jax import lax
from jax.experimental import pallas as pl
from jax.experimental.pallas import tpu as pltpu
from jax.experimental.pallas import tpu_sc as plsc

_MASKING_RATIO = 0.75


def _lex_lt(av, ai, bv, bi):
    return (av < bv) | ((av == bv) & (ai < bi))


def _substage(val, key_i, pos, j, k):
    low = (pos & j) == 0
    pval = jnp.where(low, jnp.roll(val, -j, axis=1), jnp.roll(val, j, axis=1))
    pidx = jnp.where(low, jnp.roll(key_i, -j, axis=1),
                     jnp.roll(key_i, j, axis=1))
    lt = _lex_lt(val, key_i, pval, pidx)
    up = (pos & k) == 0
    take_self = lt == (low == up)
    return jnp.where(take_self, val, pval), jnp.where(take_self, key_i, pidx)


def _sort_mask_body(noise_ref, gids_ref, mask_ref):
    n, l = noise_ref.shape
    keep = gids_ref.shape[1]
    orig = noise_ref[...]
    pos = lax.broadcasted_iota(jnp.int32, (n, l), 1)
    val = orig
    key_i = pos
    k = 2
    while k < l:
        j = k // 2
        while j >= 1:
            val, key_i = _substage(val, key_i, pos, j, k)
            j //= 2
        k *= 2
    j = l // 2
    width = l
    while width > keep:
        half = width // 2
        lo_v, hi_v = val[:, :half], val[:, half:]
        lo_i, hi_i = key_i[:, :half], key_i[:, half:]
        swap = _lex_lt(hi_v, hi_i, lo_v, lo_i)
        val = jnp.where(swap, hi_v, lo_v)
        key_i = jnp.where(swap, hi_i, lo_i)
        if half == keep:
            disc_v = jnp.where(swap, lo_v, hi_v)
            disc_i = jnp.where(swap, lo_i, hi_i)
        width = half
    pos_k = lax.broadcasted_iota(jnp.int32, (n, keep), 1)
    j = keep // 2
    while j >= 1:
        val, key_i = _substage(val, key_i, pos_k, j, 2 * keep)
        j //= 2
    w = keep
    while w > 1:
        h = w // 2
        a_v, b_v = disc_v[:, :h], disc_v[:, h:w]
        a_i, b_i = disc_i[:, :h], disc_i[:, h:w]
        t = _lex_lt(b_v, b_i, a_v, a_i)
        disc_v = jnp.where(t, b_v, a_v)
        disc_i = jnp.where(t, b_i, a_i)
        w = h
    row = lax.broadcasted_iota(jnp.int32, (n, keep), 0)
    gids_ref[...] = key_i + row * l
    tv = disc_v[:, :1]
    ti = disc_i[:, :1]
    mask = (orig > tv) | ((orig == tv) & (pos >= ti))
    mask_ref[...] = mask.astype(mask_ref.dtype)


def _sort_mask(noise, keep):
    n, l = noise.shape
    return pl.pallas_call(
        _sort_mask_body,
        out_shape=(
            jax.ShapeDtypeStruct((n, keep), jnp.int32),
            jax.ShapeDtypeStruct((n, l), jnp.float32),
        ),
    )(noise)


def _make_sc_gather(n, keep, d, chunk):
    rows = n * keep
    info = plsc.get_sparse_core_info()
    nc, ns = info.num_cores, info.num_subcores
    nw = nc * ns
    per_w = rows // nw
    rows_per_w = per_w // keep
    assert per_w % chunk == 0 and per_w % 8 == 0 and chunk % 8 == 0
    assert per_w % keep == 0 and keep % chunk == 0
    n_chunks = per_w // chunk
    mesh = plsc.VectorSubcoreMesh(core_axis_name="c", subcore_axis_name="s")

    @functools.partial(
        pl.kernel,
        mesh=mesh,
        out_type=jax.ShapeDtypeStruct((rows, d), jnp.float32),
        scratch_types=(
            [pltpu.VMEM((per_w,), jnp.int32)]
            + [pltpu.VMEM((chunk, d), jnp.float32) for _ in range(4)]
            + [pltpu.SemaphoreType.DMA for _ in range(8)]
        ),
    )
    def gather(table_hbm, gids_hbm, out_hbm, idx_all, *bufsem):
        nbuf = 4
        bufs = bufsem[:nbuf]
        gsem = bufsem[nbuf:2 * nbuf]
        wsem = bufsem[2 * nbuf:3 * nbuf]
        wid = lax.axis_index("s") * nc + lax.axis_index("c")
        base = wid * per_w
        for r in range(rows_per_w):
            pltpu.sync_copy(gids_hbm.at[wid * rows_per_w + r],
                            idx_all.at[pl.ds(r * keep, keep)])

        def start(c):
            return pltpu.async_copy(
                table_hbm.at[idx_all.at[pl.ds(c * chunk, chunk)]],
                bufs[c % nbuf], gsem[c % nbuf])

        def wback(c):
            return pltpu.async_copy(
                bufs[c % nbuf], out_hbm.at[pl.ds(base + c * chunk, chunk)],
                wsem[c % nbuf])

        gcp = [None] * n_chunks
        wcp = [None] * n_chunks
        for c in range(min(nbuf, n_chunks)):
            gcp[c] = start(c)
        for c in range(n_chunks):
            gcp[c].wait()
            wcp[c] = wback(c)
            if c + nbuf < n_chunks:
                wcp[c].wait()
                gcp[c + nbuf] = start(c + nbuf)
        for c in range(max(0, n_chunks - nbuf), n_chunks):
            wcp[c].wait()

    return gather


def kernel(x, noise):
    n, l, d = x.shape
    keep = int(l * (1 - _MASKING_RATIO))
    gids, mask = _sort_mask(noise, keep)
    table = x.reshape(n * l, d)
    gather = _make_sc_gather(n, keep, d, chunk=32)
    x_masked = gather(table, gids)
    return x_masked.reshape(n, keep, d), mask
